# baseline (speedup 1.0000x reference)
import functools

import jax
import jax.numpy as jnp
from jax import lax
from jax.experimental import pallas as pl
from jax.experimental.pallas import tpu as pltpu
from jax.experimental.pallas import tpu_sc as plsc

N = 10000
E = 320000
D = 128
G = 128

NC = 2
NS = 16
NT = NC * NS
EPT = E // NT
K = 128
EPP = 10240
NCHUNK = EPP // K
NPAIR = NCHUNK // 2
NP = 10240
RPT = NP // NS
ZR = 128

BN = 1000
NBLK = N // BN


def _sc_mesh():
    return plsc.VectorSubcoreMesh(core_axis_name="c", subcore_axis_name="s")


@functools.partial(
    pl.kernel,
    mesh=_sc_mesh(),
    out_type=jax.ShapeDtypeStruct((NC, NP, 16), jnp.float32),
    scratch_types=[
        pltpu.VMEM((NCHUNK, K), jnp.int32),
        pltpu.VMEM((K, 16), jnp.float32),
        pltpu.VMEM((ZR, 16), jnp.float32),
        pltpu.VMEM_SHARED((NP, 16), jnp.float32),
    ],
)
def _sc_degree(dst_hbm, out_hbm, dsts, ones_v, zbuf, deg_sh):
    c = lax.axis_index("c")
    s = lax.axis_index("s")
    t = c * NS + s

    pltpu.sync_copy(dst_hbm.at[t], dsts)

    def fill(i, _):
        ones_v[i, :] = jnp.ones((16,), jnp.float32)
        return 0

    lax.fori_loop(0, K, fill, 0)

    def fill2(i, _):
        zbuf[i, :] = jnp.zeros((16,), jnp.float32)
        return 0

    lax.fori_loop(0, ZR, fill2, 0)
    for i in range(RPT // ZR):
        pltpu.sync_copy(zbuf, deg_sh.at[pl.ds(s * RPT + i * ZR, ZR)])
    plsc.subcore_barrier()

    def chunk(j, _):
        pltpu.sync_copy(ones_v, deg_sh.at[dsts.at[j]], add=True)
        return 0

    lax.fori_loop(0, NCHUNK, chunk, 0)
    plsc.subcore_barrier()
    pltpu.sync_copy(deg_sh.at[pl.ds(s * RPT, RPT)],
                    out_hbm.at[c, pl.ds(s * RPT, RPT)])


@functools.partial(
    pl.kernel,
    mesh=_sc_mesh(),
    out_type=jax.ShapeDtypeStruct((NC, NP, D), jnp.float32),
    scratch_types=[
        pltpu.VMEM((K,), jnp.int32),
        pltpu.VMEM((K,), jnp.int32),
        pltpu.VMEM((K, D), jnp.float32),
        pltpu.VMEM_SHARED((NP, D), jnp.float32),
        pltpu.SemaphoreType.DMA,
    ],
)
def _sc_msgpass(h_hbm, src_hbm, dst_hbm, out_hbm,
                srcv, dstv0, rows0, acc_sh, gs0):
    c = lax.axis_index("c")
    s = lax.axis_index("s")
    t = c * NS + s

    def zb(i, _):
        for j in range(D // 16):
            rows0[i, pl.ds(j * 16, 16)] = jnp.zeros((16,), jnp.float32)
        return 0

    lax.fori_loop(0, K, zb, 0)
    for i in range(RPT // K):
        pltpu.sync_copy(rows0, acc_sh.at[pl.ds(s * RPT + i * K, K)])
    plsc.subcore_barrier()

    def chunk(j, _):
        pltpu.sync_copy(src_hbm.at[t, pl.ds(j * K, K)], srcv)
        pltpu.sync_copy(dst_hbm.at[t, pl.ds(j * K, K)], dstv0)
        pltpu.async_copy(h_hbm.at[srcv], rows0, gs0).wait()
        pltpu.sync_copy(rows0, acc_sh.at[dstv0], add=True)
        return 0

    lax.fori_loop(0, NCHUNK, chunk, 0)
    plsc.subcore_barrier()
    pltpu.sync_copy(acc_sh.at[pl.ds(s * RPT, RPT)],
                    out_hbm.at[c, pl.ds(s * RPT, RPT)])


def _t1_body(degp_ref, x_ref, w_ref, h1p_ref, dinv_ref):
    dsum = degp_ref[0, :, 0:1] + degp_ref[1, :, 0:1] + 1.0
    dinv = lax.rsqrt(dsum)
    h = jnp.dot(x_ref[...], w_ref[...], preferred_element_type=jnp.float32)
    h1p_ref[...] = h * dinv
    dinv_ref[...] = dinv


def _t1(degp, x, W1):
    return pl.pallas_call(
        _t1_body,
        grid=(NBLK,),
        in_specs=[
            pl.BlockSpec((NC, BN, 16), lambda i: (0, i, 0)),
            pl.BlockSpec((BN, D), lambda i: (i, 0)),
            pl.BlockSpec((D, D), lambda i: (0, 0)),
        ],
        out_specs=[
            pl.BlockSpec((BN, D), lambda i: (i, 0)),
            pl.BlockSpec((BN, 1), lambda i: (i, 0)),
        ],
        out_shape=[
            jax.ShapeDtypeStruct((N, D), jnp.float32),
            jax.ShapeDtypeStruct((N, 1), jnp.float32),
        ],
    )(degp, x, W1)


def _t2_body(sp_ref, h1p_ref, dinv_ref, x_ref, b1_ref, w2_ref,
             a1_ref, h2p_ref):
    S = sp_ref[0] + sp_ref[1]
    dinv = dinv_ref[...]
    out1 = dinv * (S + h1p_ref[...]) + b1_ref[...]
    a1 = jnp.maximum(out1, 0.0) + x_ref[...]
    a1_ref[...] = a1
    h2p_ref[...] = jnp.dot(
        a1, w2_ref[...], preferred_element_type=jnp.float32) * dinv


def _t2(s1, h1p, dinv, x, b1, W2):
    return pl.pallas_call(
        _t2_body,
        grid=(NBLK,),
        in_specs=[
            pl.BlockSpec((NC, BN, D), lambda i: (0, i, 0)),
            pl.BlockSpec((BN, D), lambda i: (i, 0)),
            pl.BlockSpec((BN, 1), lambda i: (i, 0)),
            pl.BlockSpec((BN, D), lambda i: (i, 0)),
            pl.BlockSpec((1, D), lambda i: (0, 0)),
            pl.BlockSpec((D, D), lambda i: (0, 0)),
        ],
        out_specs=[
            pl.BlockSpec((BN, D), lambda i: (i, 0)),
            pl.BlockSpec((BN, D), lambda i: (i, 0)),
        ],
        out_shape=[
            jax.ShapeDtypeStruct((N, D), jnp.float32),
            jax.ShapeDtypeStruct((N, D), jnp.float32),
        ],
    )(s1, h1p, dinv, x, b1, W2)


def _t3_body(sp_ref, h2p_ref, dinv_ref, a1_ref, b2_ref,
             batch_c_ref, mW1_ref, mb1_ref, mW2_ref, mb2_ref,
             out_ref, maxp_acc, addp_acc):
    i = pl.program_id(0)
    S = sp_ref[0] + sp_ref[1]
    dinv = dinv_ref[...]
    out2 = dinv * (S + h2p_ref[...]) + b2_ref[...]
    a2 = jnp.maximum(out2, 0.0) + a1_ref[...]

    @pl.when(i == 0)
    def _():
        addp_acc[...] = jnp.zeros_like(addp_acc)
        maxp_acc[...] = jnp.full_like(maxp_acc, -jnp.inf)

    bc = batch_c_ref[...]
    gid = lax.broadcasted_iota(jnp.int32, (BN, G), 1)
    mask = (gid == bc).astype(jnp.float32)
    addp_acc[...] += lax.dot_general(
        mask, a2, (((0,), (0,)), ((), ())),
        preferred_element_type=jnp.float32)

    g_lo = bc[0, 0]
    g_hi = bc[BN - 1, 0]

    def mbody(g, _):
        m = bc == g
        v = jnp.where(m, a2, -jnp.inf)
        vmax = jnp.max(v, axis=0, keepdims=True)
        cur = maxp_acc[pl.ds(g, 1), :]
        maxp_acc[pl.ds(g, 1), :] = jnp.maximum(cur, vmax)
        return 0

    lax.fori_loop(g_lo, g_hi + 1, mbody, 0)

    @pl.when(i == NBLK - 1)
    def _():
        z = jnp.concatenate([maxp_acc[...], addp_acc[...]], axis=1)
        zz = jnp.dot(z, mW1_ref[...], preferred_element_type=jnp.float32)
        zz = jnp.maximum(zz + mb1_ref[...], 0.0)
        out_ref[...] = jnp.dot(
            zz, mW2_ref[...], preferred_element_type=jnp.float32) + mb2_ref[...]


def _t3(s2, h2p, dinv, a1, b2, batch_c, mW1, mb1, mW2, mb2):
    return pl.pallas_call(
        _t3_body,
        grid=(NBLK,),
        in_specs=[
            pl.BlockSpec((NC, BN, D), lambda i: (0, i, 0)),
            pl.BlockSpec((BN, D), lambda i: (i, 0)),
            pl.BlockSpec((BN, 1), lambda i: (i, 0)),
            pl.BlockSpec((BN, D), lambda i: (i, 0)),
            pl.BlockSpec((1, D), lambda i: (0, 0)),
            pl.BlockSpec((BN, 1), lambda i: (i, 0)),
            pl.BlockSpec((2 * D, D), lambda i: (0, 0)),
            pl.BlockSpec((1, D), lambda i: (0, 0)),
            pl.BlockSpec((D, D), lambda i: (0, 0)),
            pl.BlockSpec((1, D), lambda i: (0, 0)),
        ],
        out_specs=pl.BlockSpec((G, D), lambda i: (0, 0)),
        out_shape=jax.ShapeDtypeStruct((G, D), jnp.float32),
        scratch_shapes=[
            pltpu.VMEM((G, D), jnp.float32),
            pltpu.VMEM((G, D), jnp.float32),
        ],
    )(s2, h2p, dinv, a1, b2, batch_c, mW1, mb1, mW2, mb2)


def kernel(x, edge_index, edge_attr, batch, W1, b1, W2, b2,
           mW1, mb1, mW2, mb2):
    srcp = jnp.pad(edge_index[0].reshape(NT, EPT), ((0, 0), (0, EPP - EPT)))
    dstp = jnp.pad(edge_index[1].reshape(NT, EPT), ((0, 0), (0, EPP - EPT)),
                   constant_values=N)
    dst3 = dstp.reshape(NT, NCHUNK, K)
    batch_c = batch.reshape(N, 1)

    degp = _sc_degree(dst3)
    h1p, dinv = _t1(degp, x, W1)
    s1 = _sc_msgpass(h1p, srcp, dstp)
    a1, h2p = _t2(s1, h1p, dinv, x, b1.reshape(1, D), W2)
    s2 = _sc_msgpass(h2p, srcp, dstp)
    return _t3(s2, h2p, dinv, a1, b2.reshape(1, D), batch_c,
               mW1, mb1.reshape(1, D), mW2, mb2.reshape(1, D))


# speedup vs baseline: 1.5811x; 1.5811x over previous
import functools

import jax
import jax.numpy as jnp
from jax import lax
from jax.experimental import pallas as pl
from jax.experimental.pallas import tpu as pltpu
from jax.experimental.pallas import tpu_sc as plsc

N = 10000
E = 320000
D = 128
G = 128

NC = 2
NS = 16
NT = NC * NS
EPT = E // NT
KD = 128
EPP = 10240
NCHD = EPP // KD
K = 80
NCHUNK = EPT // K
NP = 10240
RPT = NP // NS
ZR = 128

BN = 1000
NBLK = N // BN


def _sc_mesh():
    return plsc.VectorSubcoreMesh(core_axis_name="c", subcore_axis_name="s")


@functools.partial(
    pl.kernel,
    mesh=_sc_mesh(),
    out_type=jax.ShapeDtypeStruct((NC, NP, 16), jnp.float32),
    scratch_types=[
        pltpu.VMEM((NCHD, KD), jnp.int32),
        pltpu.VMEM((KD, 16), jnp.float32),
        pltpu.VMEM((ZR, 16), jnp.float32),
        pltpu.VMEM_SHARED((NP, 16), jnp.float32),
    ],
)
def _sc_degree(dst_hbm, out_hbm, dsts, ones_v, zbuf, deg_sh):
    c = lax.axis_index("c")
    s = lax.axis_index("s")
    t = c * NS + s

    pltpu.sync_copy(dst_hbm.at[t], dsts)

    def fill(i, _):
        ones_v[i, :] = jnp.ones((16,), jnp.float32)
        return 0

    lax.fori_loop(0, KD, fill, 0)

    def fill2(i, _):
        zbuf[i, :] = jnp.zeros((16,), jnp.float32)
        return 0

    lax.fori_loop(0, ZR, fill2, 0)
    for i in range(RPT // ZR):
        pltpu.sync_copy(zbuf, deg_sh.at[pl.ds(s * RPT + i * ZR, ZR)])
    plsc.subcore_barrier()

    def chunk(j, _):
        pltpu.sync_copy(ones_v, deg_sh.at[dsts.at[j]], add=True)
        return 0

    lax.fori_loop(0, NCHD, chunk, 0)
    plsc.subcore_barrier()
    pltpu.sync_copy(deg_sh.at[pl.ds(s * RPT, RPT)],
                    out_hbm.at[c, pl.ds(s * RPT, RPT)])


@functools.partial(
    pl.kernel,
    mesh=_sc_mesh(),
    out_type=jax.ShapeDtypeStruct((NC, NP, D), jnp.float32),
    scratch_types=[
        pltpu.VMEM((K,), jnp.int32),
        pltpu.VMEM((K,), jnp.int32),
        pltpu.VMEM((K,), jnp.int32),
        pltpu.VMEM((K,), jnp.int32),
        pltpu.VMEM((K, D), jnp.float32),
        pltpu.VMEM((K, D), jnp.float32),
        pltpu.VMEM_SHARED((NP, D), jnp.float32),
        pltpu.SemaphoreType.DMA,
        pltpu.SemaphoreType.DMA,
    ],
)
def _sc_msgpass(h_hbm, src_hbm, dst_hbm, out_hbm,
                srcv0, dstv0, srcv1, dstv1, rows0, rows1, acc_sh, gs0, gs1):
    c = lax.axis_index("c")
    s = lax.axis_index("s")
    t = c * NS + s

    def zb(i, _):
        for j in range(D // 16):
            rows0[i, pl.ds(j * 16, 16)] = jnp.zeros((16,), jnp.float32)
        return 0

    lax.fori_loop(0, K, zb, 0)
    for i in range(RPT // K):
        pltpu.sync_copy(rows0, acc_sh.at[pl.ds(s * RPT + i * K, K)])
    plsc.subcore_barrier()

    def one(base, srcv, dstv, rows, gs):
        pltpu.sync_copy(src_hbm.at[pl.ds(base, K)], srcv)
        pltpu.sync_copy(dst_hbm.at[pl.ds(base, K)], dstv)
        pltpu.async_copy(h_hbm.at[srcv], rows, gs).wait()
        pltpu.sync_copy(rows, acc_sh.at[dstv], add=True)

    def pair(i, _):
        base = t * EPT + 2 * i * K
        one(base, srcv0, dstv0, rows0, gs0)
        one(base + K, srcv1, dstv1, rows1, gs1)
        return 0

    lax.fori_loop(0, NCHUNK // 2, pair, 0)
    one(t * EPT + (NCHUNK - 1) * K, srcv0, dstv0, rows0, gs0)
    plsc.subcore_barrier()
    pltpu.sync_copy(acc_sh.at[pl.ds(s * RPT, RPT)],
                    out_hbm.at[c, pl.ds(s * RPT, RPT)])


def _t1_body(degp_ref, x_ref, w_ref, h1p_ref, dinv_ref):
    dsum = degp_ref[0, :, 0:1] + degp_ref[1, :, 0:1] + 1.0
    dinv = lax.rsqrt(dsum)
    h = jnp.dot(x_ref[...], w_ref[...], preferred_element_type=jnp.float32)
    h1p_ref[...] = h * dinv
    dinv_ref[...] = dinv


def _t1(degp, x, W1):
    return pl.pallas_call(
        _t1_body,
        grid=(NBLK,),
        in_specs=[
            pl.BlockSpec((NC, BN, 16), lambda i: (0, i, 0)),
            pl.BlockSpec((BN, D), lambda i: (i, 0)),
            pl.BlockSpec((D, D), lambda i: (0, 0)),
        ],
        out_specs=[
            pl.BlockSpec((BN, D), lambda i: (i, 0)),
            pl.BlockSpec((BN, 1), lambda i: (i, 0)),
        ],
        out_shape=[
            jax.ShapeDtypeStruct((N, D), jnp.float32),
            jax.ShapeDtypeStruct((N, 1), jnp.float32),
        ],
    )(degp, x, W1)


def _t2_body(sp_ref, h1p_ref, dinv_ref, x_ref, b1_ref, w2_ref,
             a1_ref, h2p_ref):
    S = sp_ref[0] + sp_ref[1]
    dinv = dinv_ref[...]
    out1 = dinv * (S + h1p_ref[...]) + b1_ref[...]
    a1 = jnp.maximum(out1, 0.0) + x_ref[...]
    a1_ref[...] = a1
    h2p_ref[...] = jnp.dot(
        a1, w2_ref[...], preferred_element_type=jnp.float32) * dinv


def _t2(s1, h1p, dinv, x, b1, W2):
    return pl.pallas_call(
        _t2_body,
        grid=(NBLK,),
        in_specs=[
            pl.BlockSpec((NC, BN, D), lambda i: (0, i, 0)),
            pl.BlockSpec((BN, D), lambda i: (i, 0)),
            pl.BlockSpec((BN, 1), lambda i: (i, 0)),
            pl.BlockSpec((BN, D), lambda i: (i, 0)),
            pl.BlockSpec((1, D), lambda i: (0, 0)),
            pl.BlockSpec((D, D), lambda i: (0, 0)),
        ],
        out_specs=[
            pl.BlockSpec((BN, D), lambda i: (i, 0)),
            pl.BlockSpec((BN, D), lambda i: (i, 0)),
        ],
        out_shape=[
            jax.ShapeDtypeStruct((N, D), jnp.float32),
            jax.ShapeDtypeStruct((N, D), jnp.float32),
        ],
    )(s1, h1p, dinv, x, b1, W2)


def _t3_body(sp_ref, h2p_ref, dinv_ref, a1_ref, b2_ref,
             batch_c_ref, mW1_ref, mb1_ref, mW2_ref, mb2_ref,
             out_ref, maxp_acc, addp_acc):
    i = pl.program_id(0)
    S = sp_ref[0] + sp_ref[1]
    dinv = dinv_ref[...]
    out2 = dinv * (S + h2p_ref[...]) + b2_ref[...]
    a2 = jnp.maximum(out2, 0.0) + a1_ref[...]

    @pl.when(i == 0)
    def _():
        addp_acc[...] = jnp.zeros_like(addp_acc)
        maxp_acc[...] = jnp.full_like(maxp_acc, -jnp.inf)

    bc = batch_c_ref[...]
    gid = lax.broadcasted_iota(jnp.int32, (BN, G), 1)
    mask = (gid == bc).astype(jnp.float32)
    addp_acc[...] += lax.dot_general(
        mask, a2, (((0,), (0,)), ((), ())),
        preferred_element_type=jnp.float32)

    g_lo = bc[0, 0]
    g_hi = bc[BN - 1, 0]

    def mbody(g, _):
        m = bc == g
        v = jnp.where(m, a2, -jnp.inf)
        vmax = jnp.max(v, axis=0, keepdims=True)
        cur = maxp_acc[pl.ds(g, 1), :]
        maxp_acc[pl.ds(g, 1), :] = jnp.maximum(cur, vmax)
        return 0

    lax.fori_loop(g_lo, g_hi + 1, mbody, 0)

    @pl.when(i == NBLK - 1)
    def _():
        z = jnp.concatenate([maxp_acc[...], addp_acc[...]], axis=1)
        zz = jnp.dot(z, mW1_ref[...], preferred_element_type=jnp.float32)
        zz = jnp.maximum(zz + mb1_ref[...], 0.0)
        out_ref[...] = jnp.dot(
            zz, mW2_ref[...], preferred_element_type=jnp.float32) + mb2_ref[...]


def _t3(s2, h2p, dinv, a1, b2, batch_c, mW1, mb1, mW2, mb2):
    return pl.pallas_call(
        _t3_body,
        grid=(NBLK,),
        in_specs=[
            pl.BlockSpec((NC, BN, D), lambda i: (0, i, 0)),
            pl.BlockSpec((BN, D), lambda i: (i, 0)),
            pl.BlockSpec((BN, 1), lambda i: (i, 0)),
            pl.BlockSpec((BN, D), lambda i: (i, 0)),
            pl.BlockSpec((1, D), lambda i: (0, 0)),
            pl.BlockSpec((BN, 1), lambda i: (i, 0)),
            pl.BlockSpec((2 * D, D), lambda i: (0, 0)),
            pl.BlockSpec((1, D), lambda i: (0, 0)),
            pl.BlockSpec((D, D), lambda i: (0, 0)),
            pl.BlockSpec((1, D), lambda i: (0, 0)),
        ],
        out_specs=pl.BlockSpec((G, D), lambda i: (0, 0)),
        out_shape=jax.ShapeDtypeStruct((G, D), jnp.float32),
        scratch_shapes=[
            pltpu.VMEM((G, D), jnp.float32),
            pltpu.VMEM((G, D), jnp.float32),
        ],
    )(s2, h2p, dinv, a1, b2, batch_c, mW1, mb1, mW2, mb2)


def kernel(x, edge_index, edge_attr, batch, W1, b1, W2, b2,
           mW1, mb1, mW2, mb2):
    dst2 = edge_index[1].reshape(NT, EPT)
    dst3d = jnp.pad(dst2, ((0, 0), (0, EPP - EPT)),
                    constant_values=N).reshape(NT, NCHD, KD)
    batch_c = batch.reshape(N, 1)

    src1 = edge_index[0]
    dst1 = edge_index[1]
    degp = _sc_degree(dst3d)
    h1p, dinv = _t1(degp, x, W1)
    s1 = _sc_msgpass(h1p, src1, dst1)
    a1, h2p = _t2(s1, h1p, dinv, x, b1.reshape(1, D), W2)
    s2 = _sc_msgpass(h2p, src1, dst1)
    return _t3(s2, h2p, dinv, a1, b2.reshape(1, D), batch_c,
               mW1, mb1.reshape(1, D), mW2, mb2.reshape(1, D))
